# Initial kernel scaffold; baseline (speedup 1.0000x reference)
import functools

import jax
import jax.numpy as jnp
from jax import lax
from jax.experimental import pallas as pl
from jax.experimental.pallas import tpu as pltpu

N = 10000
E = 320000
IN_FEATS = 128
HID = 16
HEADS = 8
OUT = 16

ROWS = 1000


def _t1_body(h_ref, w_ref, al_ref, ar_ref, feat_ref, el_ref, er_ref):
    feat = jnp.dot(h_ref[...], w_ref[...], preferred_element_type=jnp.float32)
    feat_ref[...] = feat
    el_ref[...] = jnp.dot(feat, al_ref[...], preferred_element_type=jnp.float32)
    er_ref[...] = jnp.dot(feat, ar_ref[...], preferred_element_type=jnp.float32)


def _t1(h, W1, al, ar):
    grid = (N // ROWS,)
    return pl.pallas_call(
        _t1_body,
        grid=grid,
        in_specs=[
            pl.BlockSpec((ROWS, IN_FEATS), lambda i: (i, 0)),
            pl.BlockSpec((IN_FEATS, HEADS * HID), lambda i: (0, 0)),
            pl.BlockSpec((IN_FEATS, HEADS), lambda i: (0, 0)),
            pl.BlockSpec((IN_FEATS, HEADS), lambda i: (0, 0)),
        ],
        out_specs=[
            pl.BlockSpec((ROWS, HEADS * HID), lambda i: (i, 0)),
            pl.BlockSpec((ROWS, HEADS), lambda i: (i, 0)),
            pl.BlockSpec((ROWS, HEADS), lambda i: (i, 0)),
        ],
        out_shape=[
            jax.ShapeDtypeStruct((N, HEADS * HID), jnp.float32),
            jax.ShapeDtypeStruct((N, HEADS), jnp.float32),
            jax.ShapeDtypeStruct((N, HEADS), jnp.float32),
        ],
    )(h, W1, al, ar)


def _t2_body(x_ref, b_ref, w_ref, alr_ref, feat_ref, eler_ref):
    h1 = x_ref[...] + b_ref[...]
    h1 = jnp.where(h1 > 0, h1, jnp.expm1(h1))
    feat = jnp.dot(h1, w_ref[...], preferred_element_type=jnp.float32)
    feat_ref[...] = feat
    eler_ref[...] = jnp.dot(feat, alr_ref[...], preferred_element_type=jnp.float32)


def _t2(out1, b1, W2, alr2):
    grid = (N // ROWS,)
    return pl.pallas_call(
        _t2_body,
        grid=grid,
        in_specs=[
            pl.BlockSpec((ROWS, HEADS * HID), lambda i: (i, 0)),
            pl.BlockSpec((1, HEADS * HID), lambda i: (0, 0)),
            pl.BlockSpec((HEADS * HID, OUT), lambda i: (0, 0)),
            pl.BlockSpec((OUT, 2), lambda i: (0, 0)),
        ],
        out_specs=[
            pl.BlockSpec((ROWS, OUT), lambda i: (i, 0)),
            pl.BlockSpec((ROWS, 2), lambda i: (i, 0)),
        ],
        out_shape=[
            jax.ShapeDtypeStruct((N, OUT), jnp.float32),
            jax.ShapeDtypeStruct((N, 2), jnp.float32),
        ],
    )(out1, b1, W2, alr2)


def _t3_body(x_ref, b_ref, o_ref):
    x = x_ref[...] + b_ref[...]
    m = jnp.max(x, axis=1, keepdims=True)
    ex = jnp.exp(x - m)
    o_ref[...] = ex / jnp.sum(ex, axis=1, keepdims=True)


def _t3(out2, b2):
    grid = (N // ROWS,)
    return pl.pallas_call(
        _t3_body,
        grid=grid,
        in_specs=[
            pl.BlockSpec((ROWS, OUT), lambda i: (i, 0)),
            pl.BlockSpec((1, OUT), lambda i: (0, 0)),
        ],
        out_specs=pl.BlockSpec((ROWS, OUT), lambda i: (i, 0)),
        out_shape=jax.ShapeDtypeStruct((N, OUT), jnp.float32),
    )(out2, b2)


def _edge_phase(feat, el, er, src, dst):
    H = el.shape[1]
    D = feat.shape[1] // H
    e = jax.nn.leaky_relu(el[src] + er[dst], negative_slope=0.2)
    p = jnp.exp(e)
    s = jax.ops.segment_sum(p, dst, num_segments=N)
    a = p / (s[dst] + 1e-9)
    msg = feat[src].reshape(E, H, D) * a[:, :, None]
    out = jax.ops.segment_sum(msg.reshape(E, H * D), dst, num_segments=N)
    return a, out


def _block_diag_lr(al, ar):
    H, D = al.shape
    eye = jnp.eye(H, dtype=al.dtype)
    ml = (al[:, :, None] * eye[:, None, :]).reshape(H * D, H)
    mr = (ar[:, :, None] * eye[:, None, :]).reshape(H * D, H)
    return ml, mr


def kernel(h, edge_index, W1, al1, ar1, b1, W2, al2, ar2, b2):
    src = edge_index[0]
    dst = edge_index[1]

    ml1, mr1 = _block_diag_lr(al1, ar1)
    feat1, el1, er1 = _t1(h, W1, ml1, mr1)
    a1, out1 = _edge_phase(feat1, el1, er1, src, dst)

    alr2 = jnp.concatenate([al2.T, ar2.T], axis=1)
    feat2, eler2 = _t2(out1, b1.reshape(1, -1), W2, alr2)
    a2, out2 = _edge_phase(feat2, eler2[:, :1], eler2[:, 1:], src, dst)

    out = _t3(out2, b2.reshape(1, -1))
    return out, [a1[:, :, None], a2[:, :, None]]


# speedup vs baseline: 4.4153x; 4.4153x over previous
import functools

import jax
import jax.numpy as jnp
from jax import lax
from jax.experimental import pallas as pl
from jax.experimental.pallas import tpu as pltpu

N = 10000
E = 320000
IN_FEATS = 128
HID = 16
HEADS = 8
OUT = 16

ROWS = 1000


def _t1_body(h_ref, w_ref, al_ref, ar_ref, feat_ref, el_ref, er_ref):
    feat = jnp.dot(h_ref[...], w_ref[...], preferred_element_type=jnp.float32)
    feat_ref[...] = feat
    el_ref[...] = jnp.dot(feat, al_ref[...], preferred_element_type=jnp.float32)
    er_ref[...] = jnp.dot(feat, ar_ref[...], preferred_element_type=jnp.float32)


def _t1(h, W1, al, ar):
    grid = (N // ROWS,)
    return pl.pallas_call(
        _t1_body,
        grid=grid,
        in_specs=[
            pl.BlockSpec((ROWS, IN_FEATS), lambda i: (i, 0)),
            pl.BlockSpec((IN_FEATS, HEADS * HID), lambda i: (0, 0)),
            pl.BlockSpec((IN_FEATS, HEADS), lambda i: (0, 0)),
            pl.BlockSpec((IN_FEATS, HEADS), lambda i: (0, 0)),
        ],
        out_specs=[
            pl.BlockSpec((ROWS, HEADS * HID), lambda i: (i, 0)),
            pl.BlockSpec((ROWS, HEADS), lambda i: (i, 0)),
            pl.BlockSpec((ROWS, HEADS), lambda i: (i, 0)),
        ],
        out_shape=[
            jax.ShapeDtypeStruct((N, HEADS * HID), jnp.float32),
            jax.ShapeDtypeStruct((N, HEADS), jnp.float32),
            jax.ShapeDtypeStruct((N, HEADS), jnp.float32),
        ],
    )(h, W1, al, ar)


def _t2_body(x_ref, b_ref, w_ref, alr_ref, feat_ref, eler_ref):
    h1 = x_ref[...] + b_ref[...]
    h1 = jnp.where(h1 > 0, h1, jnp.exp(jnp.minimum(h1, 0.0)) - 1.0)
    feat = jnp.dot(h1, w_ref[...], preferred_element_type=jnp.float32)
    feat_ref[...] = feat
    eler_ref[...] = jnp.dot(feat, alr_ref[...], preferred_element_type=jnp.float32)


def _t2(out1, b1, W2, alr2):
    grid = (N // ROWS,)
    return pl.pallas_call(
        _t2_body,
        grid=grid,
        in_specs=[
            pl.BlockSpec((ROWS, HEADS * HID), lambda i: (i, 0)),
            pl.BlockSpec((1, HEADS * HID), lambda i: (0, 0)),
            pl.BlockSpec((HEADS * HID, OUT), lambda i: (0, 0)),
            pl.BlockSpec((OUT, 2), lambda i: (0, 0)),
        ],
        out_specs=[
            pl.BlockSpec((ROWS, OUT), lambda i: (i, 0)),
            pl.BlockSpec((ROWS, 2), lambda i: (i, 0)),
        ],
        out_shape=[
            jax.ShapeDtypeStruct((N, OUT), jnp.float32),
            jax.ShapeDtypeStruct((N, 2), jnp.float32),
        ],
    )(out1, b1, W2, alr2)


def _t3_body(x_ref, b_ref, o_ref):
    x = x_ref[...] + b_ref[...]
    m = jnp.max(x, axis=1, keepdims=True)
    ex = jnp.exp(x - m)
    o_ref[...] = ex / jnp.sum(ex, axis=1, keepdims=True)


def _t3(out2, b2):
    grid = (N // ROWS,)
    return pl.pallas_call(
        _t3_body,
        grid=grid,
        in_specs=[
            pl.BlockSpec((ROWS, OUT), lambda i: (i, 0)),
            pl.BlockSpec((1, OUT), lambda i: (0, 0)),
        ],
        out_specs=pl.BlockSpec((ROWS, OUT), lambda i: (i, 0)),
        out_shape=jax.ShapeDtypeStruct((N, OUT), jnp.float32),
    )(out2, b2)


def _edge_phase(feat, el, er, src, dst):
    H = el.shape[1]
    D = feat.shape[1] // H
    e = jax.nn.leaky_relu(el[src] + er[dst], negative_slope=0.2)
    p = jnp.exp(e)
    s = jax.ops.segment_sum(p, dst, num_segments=N)
    a = p / (s[dst] + 1e-9)
    msg = feat[src].reshape(E, H, D) * a[:, :, None]
    out = jax.ops.segment_sum(msg.reshape(E, H * D), dst, num_segments=N)
    return a, out


def _block_diag_lr(al, ar):
    H, D = al.shape
    eye = jnp.eye(H, dtype=al.dtype)
    ml = (al[:, :, None] * eye[:, None, :]).reshape(H * D, H)
    mr = (ar[:, :, None] * eye[:, None, :]).reshape(H * D, H)
    return ml, mr


def kernel(h, edge_index, W1, al1, ar1, b1, W2, al2, ar2, b2):
    src = edge_index[0]
    dst = edge_index[1]

    ml1, mr1 = _block_diag_lr(al1, ar1)
    feat1, el1, er1 = _t1(h, W1, ml1, mr1)
    a1, out1 = _edge_phase(feat1, el1, er1, src, dst)

    alr2 = jnp.concatenate([al2.T, ar2.T], axis=1)
    feat2, eler2 = _t2(out1, b1.reshape(1, -1), W2, alr2)
    a2, out2 = _edge_phase(feat2, eler2[:, :1], eler2[:, 1:], src, dst)

    out = _t3(out2, b2.reshape(1, -1))
    return out, [a1[:, :, None], a2[:, :, None]]


# speedup vs baseline: 6.5206x; 1.4768x over previous
import functools

import jax
import jax.numpy as jnp
from jax import lax
from jax.experimental import pallas as pl
from jax.experimental.pallas import tpu as pltpu
from jax.experimental.pallas import tpu_sc as plsc

N = 10000
E = 320000
IN_FEATS = 128
HID = 16
HEADS = 8
OUT = 16
F1 = HEADS * HID

ROWS = 1000

NC, NS = 2, 16
NPAD = 10240
RPT = NPAD // NS
CH = 128
BLK = 16
E_ROWS = 2560
E_PAD = E_ROWS * CH
RPS = E_ROWS // NC
RPTE = RPS // NS
NBLK = RPTE // BLK


def _t1_body(h_ref, w_ref, al_ref, ar_ref, feat_ref, el_ref, er_ref):
    feat = jnp.dot(h_ref[...], w_ref[...], preferred_element_type=jnp.float32)
    feat_ref[...] = feat
    el_ref[...] = jnp.dot(feat, al_ref[...], preferred_element_type=jnp.float32)
    er_ref[...] = jnp.dot(feat, ar_ref[...], preferred_element_type=jnp.float32)


def _t1(h, W1, al, ar):
    return pl.pallas_call(
        _t1_body,
        grid=(N // ROWS,),
        in_specs=[
            pl.BlockSpec((ROWS, IN_FEATS), lambda i: (i, 0)),
            pl.BlockSpec((IN_FEATS, F1), lambda i: (0, 0)),
            pl.BlockSpec((IN_FEATS, HEADS), lambda i: (0, 0)),
            pl.BlockSpec((IN_FEATS, HEADS), lambda i: (0, 0)),
        ],
        out_specs=[
            pl.BlockSpec((ROWS, F1), lambda i: (i, 0)),
            pl.BlockSpec((ROWS, HEADS), lambda i: (i, 0)),
            pl.BlockSpec((ROWS, HEADS), lambda i: (i, 0)),
        ],
        out_shape=[
            jax.ShapeDtypeStruct((N, F1), jnp.float32),
            jax.ShapeDtypeStruct((N, HEADS), jnp.float32),
            jax.ShapeDtypeStruct((N, HEADS), jnp.float32),
        ],
    )(h, W1, al, ar)


def _s1a(src2d, dst2d, elT, erT, feat_p, z128, z1):
    mesh = plsc.VectorSubcoreMesh(
        core_axis_name="c", subcore_axis_name="s", num_cores=NC, num_subcores=NS)

    @functools.partial(
        pl.kernel,
        mesh=mesh,
        out_type=[
            jax.ShapeDtypeStruct((NC, NPAD, F1), jnp.float32),
            jax.ShapeDtypeStruct((NC, HEADS, NPAD), jnp.float32),
        ],
        scratch_types=[
            pltpu.VMEM((BLK, CH), jnp.int32),
            pltpu.VMEM((BLK, CH), jnp.int32),
            pltpu.VMEM((HEADS * CH,), jnp.float32),
            pltpu.VMEM((HEADS * CH,), jnp.float32),
            pltpu.VMEM((HEADS * CH,), jnp.float32),
            pltpu.VMEM((CH, F1), jnp.float32),
        ] + [pltpu.VMEM_SHARED((NPAD,), jnp.float32)] * 24
        + [
            pltpu.VMEM_SHARED((NPAD, F1), jnp.float32),
            pltpu.SemaphoreType.DMA,
            pltpu.SemaphoreType.DMA,
        ],
    )
    def k(src_h, dst_h, elT_h, erT_h, feat_h, z128_h, z1_h, outraw, s_out,
          src_v, dst_v, el_b, er_b, p_b, fr,
          e0, e1, e2, e3, e4, e5, e6, e7,
          r0_, r1, r2, r3, r4, r5, r6, r7,
          s0, s1, s2, s3, s4, s5, s6, s7,
          out_sp, gsem, fsem):
        c = lax.axis_index("c")
        t = lax.axis_index("s")
        el_sp = [e0, e1, e2, e3, e4, e5, e6, e7]
        er_sp = [r0_, r1, r2, r3, r4, r5, r6, r7]
        s_sp = [s0, s1, s2, s3, s4, s5, s6, s7]

        rr = t * RPT
        for h in range(HEADS):
            pltpu.sync_copy(elT_h.at[h, pl.ds(rr, RPT)], el_sp[h].at[pl.ds(rr, RPT)])
            pltpu.sync_copy(erT_h.at[h, pl.ds(rr, RPT)], er_sp[h].at[pl.ds(rr, RPT)])
            pltpu.sync_copy(z1_h, s_sp[h].at[pl.ds(rr, RPT)])
        pltpu.sync_copy(z128_h, out_sp.at[pl.ds(rr, RPT)])
        plsc.subcore_barrier()

        def blk_a(b, _):
            row0 = c * RPS + t * RPTE + b * BLK
            pltpu.sync_copy(src_h.at[pl.ds(row0, BLK)], src_v)
            pltpu.sync_copy(dst_h.at[pl.ds(row0, BLK)], dst_v)

            def ch_a(j, _2):
                sr = src_v.at[j]
                dr = dst_v.at[j]
                dl = []
                for h in range(HEADS):
                    dl.append(pltpu.async_copy(
                        el_sp[h].at[sr], el_b.at[pl.ds(h * CH, CH)], gsem))
                    dl.append(pltpu.async_copy(
                        er_sp[h].at[dr], er_b.at[pl.ds(h * CH, CH)], gsem))
                df = pltpu.async_copy(feat_h.at[sr], fr, fsem)
                for d in dl:
                    d.wait()
                for h in range(HEADS):
                    for kk in range(CH // 16):
                        x = (el_b[pl.ds(h * CH + kk * 16, 16)]
                             + er_b[pl.ds(h * CH + kk * 16, 16)])
                        p_b[pl.ds(h * CH + kk * 16, 16)] = jnp.exp(
                            jnp.maximum(x, x * 0.2))
                for h in range(HEADS):
                    pltpu.sync_copy(p_b.at[pl.ds(h * CH, CH)],
                                    s_sp[h].at[dr], add=True)
                df.wait()

                def mul_body(g, _3):
                    gbase = g * 16
                    for h in range(HEADS):
                        pvec = p_b[pl.ds(h * CH + gbase, 16)]
                        for i in range(16):
                            ei = gbase + i
                            fr[ei, pl.ds(h * 16, 16)] = (
                                fr[ei, pl.ds(h * 16, 16)] * pvec[i])
                    return 0

                lax.fori_loop(0, CH // 16, mul_body, 0)
                pltpu.sync_copy(fr, out_sp.at[dr], add=True)
                return 0

            lax.fori_loop(0, BLK, ch_a, 0)
            return 0

        lax.fori_loop(0, NBLK, blk_a, 0)
        plsc.subcore_barrier()

        pltpu.sync_copy(out_sp.at[pl.ds(rr, RPT)], outraw.at[c, pl.ds(rr, RPT)])
        for h in range(HEADS):
            pltpu.sync_copy(s_sp[h].at[pl.ds(rr, RPT)],
                            s_out.at[c, h, pl.ds(rr, RPT)])

    return k(src2d, dst2d, elT, erT, feat_p, z128, z1)


def _s1b(src2d, dst2d, elT, erT, sT):
    mesh = plsc.VectorSubcoreMesh(
        core_axis_name="c", subcore_axis_name="s", num_cores=NC, num_subcores=NS)

    @functools.partial(
        pl.kernel,
        mesh=mesh,
        out_type=jax.ShapeDtypeStruct((HEADS, E_PAD), jnp.float32),
        scratch_types=[
            pltpu.VMEM((BLK, CH), jnp.int32),
            pltpu.VMEM((BLK, CH), jnp.int32),
            pltpu.VMEM((HEADS * CH,), jnp.float32),
            pltpu.VMEM((HEADS * CH,), jnp.float32),
            pltpu.VMEM((HEADS * CH,), jnp.float32),
            pltpu.VMEM((HEADS * BLK * CH,), jnp.float32),
        ] + [pltpu.VMEM_SHARED((NPAD,), jnp.float32)] * 24
        + [pltpu.SemaphoreType.DMA],
    )
    def k(src_h, dst_h, elT_h, erT_h, sT_h, a_out,
          src_v, dst_v, el_b, er_b, s_b, a_st,
          e0, e1, e2, e3, e4, e5, e6, e7,
          r0_, r1, r2, r3, r4, r5, r6, r7,
          s0, s1, s2, s3, s4, s5, s6, s7,
          gsem):
        c = lax.axis_index("c")
        t = lax.axis_index("s")
        el_sp = [e0, e1, e2, e3, e4, e5, e6, e7]
        er_sp = [r0_, r1, r2, r3, r4, r5, r6, r7]
        s_sp = [s0, s1, s2, s3, s4, s5, s6, s7]

        rr = t * RPT
        for h in range(HEADS):
            pltpu.sync_copy(elT_h.at[h, pl.ds(rr, RPT)], el_sp[h].at[pl.ds(rr, RPT)])
            pltpu.sync_copy(erT_h.at[h, pl.ds(rr, RPT)], er_sp[h].at[pl.ds(rr, RPT)])
            pltpu.sync_copy(sT_h.at[h, pl.ds(rr, RPT)], s_sp[h].at[pl.ds(rr, RPT)])
        plsc.subcore_barrier()

        def blk_b(b, _):
            row0 = c * RPS + t * RPTE + b * BLK
            pltpu.sync_copy(src_h.at[pl.ds(row0, BLK)], src_v)
            pltpu.sync_copy(dst_h.at[pl.ds(row0, BLK)], dst_v)

            def ch_b(j, _2):
                sr = src_v.at[j]
                dr = dst_v.at[j]
                dl = []
                for h in range(HEADS):
                    dl.append(pltpu.async_copy(
                        el_sp[h].at[sr], el_b.at[pl.ds(h * CH, CH)], gsem))
                    dl.append(pltpu.async_copy(
                        er_sp[h].at[dr], er_b.at[pl.ds(h * CH, CH)], gsem))
                    dl.append(pltpu.async_copy(
                        s_sp[h].at[dr], s_b.at[pl.ds(h * CH, CH)], gsem))
                for d in dl:
                    d.wait()
                aoff = j * CH
                for h in range(HEADS):
                    for kk in range(CH // 16):
                        x = (el_b[pl.ds(h * CH + kk * 16, 16)]
                             + er_b[pl.ds(h * CH + kk * 16, 16)])
                        pv = jnp.exp(jnp.maximum(x, x * 0.2))
                        sv = s_b[pl.ds(h * CH + kk * 16, 16)]
                        a_st[pl.ds(h * BLK * CH + aoff + kk * 16, 16)] = (
                            pv / (sv + 1e-9))
                return 0

            lax.fori_loop(0, BLK, ch_b, 0)
            eb = (c * RPS + t * RPTE + b * BLK) * CH
            for h in range(HEADS):
                pltpu.sync_copy(a_st.at[pl.ds(h * BLK * CH, BLK * CH)],
                                a_out.at[h, pl.ds(eb, BLK * CH)])
            return 0

        lax.fori_loop(0, NBLK, blk_b, 0)

    return k(src2d, dst2d, elT, erT, sT)


def _t2_body(x_ref, s_ref, bh_ref, b_ref, w_ref, alr_ref, feat_ref, eler_ref):
    x = x_ref[0] + x_ref[1]
    s8 = s_ref[0] + s_ref[1]
    div = lax.dot_general(s8, bh_ref[...], (((0,), (0,)), ((), ())),
                          preferred_element_type=jnp.float32)
    h1 = x / (div + 1e-9) + b_ref[...]
    h1 = jnp.where(h1 > 0, h1, jnp.exp(jnp.minimum(h1, 0.0)) - 1.0)
    feat = jnp.dot(h1, w_ref[...], preferred_element_type=jnp.float32)
    feat_ref[...] = feat
    eler_ref[...] = jnp.dot(feat, alr_ref[...], preferred_element_type=jnp.float32)


def _t2(outraw, s_part, bhot, b1, W2, alr2):
    return pl.pallas_call(
        _t2_body,
        grid=(NPAD // 640,),
        in_specs=[
            pl.BlockSpec((NC, 640, F1), lambda i: (0, i, 0)),
            pl.BlockSpec((NC, HEADS, 640), lambda i: (0, 0, i)),
            pl.BlockSpec((HEADS, F1), lambda i: (0, 0)),
            pl.BlockSpec((1, F1), lambda i: (0, 0)),
            pl.BlockSpec((F1, OUT), lambda i: (0, 0)),
            pl.BlockSpec((OUT, 2), lambda i: (0, 0)),
        ],
        out_specs=[
            pl.BlockSpec((640, OUT), lambda i: (i, 0)),
            pl.BlockSpec((640, 2), lambda i: (i, 0)),
        ],
        out_shape=[
            jax.ShapeDtypeStruct((NPAD, OUT), jnp.float32),
            jax.ShapeDtypeStruct((NPAD, 2), jnp.float32),
        ],
    )(outraw, s_part, bhot, b1, W2, alr2)


def _t3_body(x_ref, b_ref, o_ref):
    x = x_ref[...] + b_ref[...]
    m = jnp.max(x, axis=1, keepdims=True)
    ex = jnp.exp(x - m)
    o_ref[...] = ex / jnp.sum(ex, axis=1, keepdims=True)


def _t3(out2, b2):
    return pl.pallas_call(
        _t3_body,
        grid=(N // ROWS,),
        in_specs=[
            pl.BlockSpec((ROWS, OUT), lambda i: (i, 0)),
            pl.BlockSpec((1, OUT), lambda i: (0, 0)),
        ],
        out_specs=pl.BlockSpec((ROWS, OUT), lambda i: (i, 0)),
        out_shape=jax.ShapeDtypeStruct((N, OUT), jnp.float32),
    )(out2, b2)


def _edge_phase(feat, el, er, src, dst):
    H = el.shape[1]
    D = feat.shape[1] // H
    e = jax.nn.leaky_relu(el[src] + er[dst], negative_slope=0.2)
    p = jnp.exp(e)
    s = jax.ops.segment_sum(p, dst, num_segments=N)
    a = p / (s[dst] + 1e-9)
    msg = feat[src].reshape(E, H, D) * a[:, :, None]
    out = jax.ops.segment_sum(msg.reshape(E, H * D), dst, num_segments=N)
    return a, out


def _block_diag_lr(al, ar):
    H, D = al.shape
    eye = jnp.eye(H, dtype=al.dtype)
    ml = (al[:, :, None] * eye[:, None, :]).reshape(H * D, H)
    mr = (ar[:, :, None] * eye[:, None, :]).reshape(H * D, H)
    return ml, mr


def kernel(h, edge_index, W1, al1, ar1, b1, W2, al2, ar2, b2):
    src = edge_index[0]
    dst = edge_index[1]

    ml1, mr1 = _block_diag_lr(al1, ar1)
    feat1, el1, er1 = _t1(h, W1, ml1, mr1)

    src_p = jnp.concatenate(
        [src, jnp.zeros((E_PAD - E,), jnp.int32)]).reshape(E_ROWS, CH)
    dst_p = jnp.concatenate(
        [dst, jnp.full((E_PAD - E,), N, jnp.int32)]).reshape(E_ROWS, CH)
    elT = jnp.pad(el1.T, ((0, 0), (0, NPAD - N)))
    erT = jnp.pad(er1.T, ((0, 0), (0, NPAD - N)))
    feat_p = jnp.pad(feat1, ((0, NPAD - N), (0, 0)))
    z128 = jnp.zeros((RPT, F1), jnp.float32)
    z1 = jnp.zeros((RPT,), jnp.float32)

    outraw, s_part = _s1a(src_p, dst_p, elT, erT, feat_p, z128, z1)
    sT = s_part[0] + s_part[1]
    a_raw = _s1b(src_p, dst_p, elT, erT, sT)
    a1 = a_raw[:, :E].T

    bhot = (jnp.eye(HEADS, dtype=jnp.float32)[:, :, None]
            * jnp.ones((1, 1, HID), jnp.float32)).reshape(HEADS, F1)
    alr2 = jnp.concatenate([al2.T, ar2.T], axis=1)
    feat2, eler2 = _t2(outraw, s_part, bhot, b1.reshape(1, -1), W2, alr2)
    a2, out2 = _edge_phase(feat2[:N], eler2[:N, :1], eler2[:N, 1:], src, dst)

    out = _t3(out2, b2.reshape(1, -1))
    return out, [a1[:, :, None], a2[:, :, None]]


# speedup vs baseline: 30.2083x; 4.6327x over previous
import functools

import jax
import jax.numpy as jnp
from jax import lax
from jax.experimental import pallas as pl
from jax.experimental.pallas import tpu as pltpu
from jax.experimental.pallas import tpu_sc as plsc

N = 10000
E = 320000
IN_FEATS = 128
HID = 16
HEADS = 8
OUT = 16
F1 = HEADS * HID

ROWS = 1000

NC, NS = 2, 16
NPAD = 10240
RPT = NPAD // NS
CH = 128
BLK = 16
E_ROWS = 2560
E_PAD = E_ROWS * CH
RPS = E_ROWS // NC
RPTE = RPS // NS
NBLK = RPTE // BLK


def _t1_body(h_ref, w_ref, al_ref, ar_ref, feat_ref, el_ref, er_ref):
    feat = jnp.dot(h_ref[...], w_ref[...], preferred_element_type=jnp.float32)
    feat_ref[...] = feat
    el_ref[...] = jnp.dot(feat, al_ref[...], preferred_element_type=jnp.float32)
    er_ref[...] = jnp.dot(feat, ar_ref[...], preferred_element_type=jnp.float32)


def _t1(h, W1, al, ar):
    return pl.pallas_call(
        _t1_body,
        grid=(N // ROWS,),
        in_specs=[
            pl.BlockSpec((ROWS, IN_FEATS), lambda i: (i, 0)),
            pl.BlockSpec((IN_FEATS, F1), lambda i: (0, 0)),
            pl.BlockSpec((IN_FEATS, HEADS), lambda i: (0, 0)),
            pl.BlockSpec((IN_FEATS, HEADS), lambda i: (0, 0)),
        ],
        out_specs=[
            pl.BlockSpec((ROWS, F1), lambda i: (i, 0)),
            pl.BlockSpec((ROWS, HEADS), lambda i: (i, 0)),
            pl.BlockSpec((ROWS, HEADS), lambda i: (i, 0)),
        ],
        out_shape=[
            jax.ShapeDtypeStruct((N, F1), jnp.float32),
            jax.ShapeDtypeStruct((N, HEADS), jnp.float32),
            jax.ShapeDtypeStruct((N, HEADS), jnp.float32),
        ],
    )(h, W1, al, ar)


def _s1a(src2d, dst2d, elT, erT, feat_p, z128, z1):
    mesh = plsc.VectorSubcoreMesh(
        core_axis_name="c", subcore_axis_name="s", num_cores=NC, num_subcores=NS)

    @functools.partial(
        pl.kernel,
        mesh=mesh,
        out_type=[
            jax.ShapeDtypeStruct((NC, NPAD, F1), jnp.float32),
            jax.ShapeDtypeStruct((NC, HEADS, NPAD), jnp.float32),
        ],
        scratch_types=[
            pltpu.VMEM((BLK, CH), jnp.int32),
            pltpu.VMEM((BLK, CH), jnp.int32),
            pltpu.VMEM((HEADS * CH,), jnp.float32),
            pltpu.VMEM((HEADS * CH,), jnp.float32),
            pltpu.VMEM((HEADS * CH,), jnp.float32),
            pltpu.VMEM((CH, F1), jnp.float32),
        ] + [pltpu.VMEM_SHARED((NPAD,), jnp.float32)] * 24
        + [
            pltpu.VMEM_SHARED((NPAD, F1), jnp.float32),
            pltpu.SemaphoreType.DMA,
            pltpu.SemaphoreType.DMA,
        ],
    )
    def k(src_h, dst_h, elT_h, erT_h, feat_h, z128_h, z1_h, outraw, s_out,
          src_v, dst_v, el_b, er_b, p_b, fr,
          e0, e1, e2, e3, e4, e5, e6, e7,
          r0_, r1, r2, r3, r4, r5, r6, r7,
          s0, s1, s2, s3, s4, s5, s6, s7,
          out_sp, gsem, fsem):
        c = lax.axis_index("c")
        t = lax.axis_index("s")
        el_sp = [e0, e1, e2, e3, e4, e5, e6, e7]
        er_sp = [r0_, r1, r2, r3, r4, r5, r6, r7]
        s_sp = [s0, s1, s2, s3, s4, s5, s6, s7]

        rr = t * RPT
        for h in range(HEADS):
            pltpu.sync_copy(elT_h.at[h, pl.ds(rr, RPT)], el_sp[h].at[pl.ds(rr, RPT)])
            pltpu.sync_copy(erT_h.at[h, pl.ds(rr, RPT)], er_sp[h].at[pl.ds(rr, RPT)])
            pltpu.sync_copy(z1_h, s_sp[h].at[pl.ds(rr, RPT)])
        pltpu.sync_copy(z128_h, out_sp.at[pl.ds(rr, RPT)])
        plsc.subcore_barrier()

        def blk_a(b, _):
            row0 = c * RPS + t * RPTE + b * BLK
            pltpu.sync_copy(src_h.at[pl.ds(row0, BLK)], src_v)
            pltpu.sync_copy(dst_h.at[pl.ds(row0, BLK)], dst_v)

            def ch_a(j, _2):
                sr = src_v.at[j]
                dr = dst_v.at[j]
                dl = []
                for h in range(HEADS):
                    dl.append(pltpu.async_copy(
                        el_sp[h].at[sr], el_b.at[pl.ds(h * CH, CH)], gsem))
                    dl.append(pltpu.async_copy(
                        er_sp[h].at[dr], er_b.at[pl.ds(h * CH, CH)], gsem))
                df = pltpu.async_copy(feat_h.at[sr], fr, fsem)
                for d in dl:
                    d.wait()
                for h in range(HEADS):
                    for kk in range(CH // 16):
                        x = (el_b[pl.ds(h * CH + kk * 16, 16)]
                             + er_b[pl.ds(h * CH + kk * 16, 16)])
                        p_b[pl.ds(h * CH + kk * 16, 16)] = jnp.exp(
                            jnp.maximum(x, x * 0.2))
                for h in range(HEADS):
                    pltpu.sync_copy(p_b.at[pl.ds(h * CH, CH)],
                                    s_sp[h].at[dr], add=True)
                df.wait()

                def mul_body(g, _3):
                    gbase = g * 16
                    for h in range(HEADS):
                        pvec = p_b[pl.ds(h * CH + gbase, 16)]
                        for i in range(16):
                            ei = gbase + i
                            fr[ei, pl.ds(h * 16, 16)] = (
                                fr[ei, pl.ds(h * 16, 16)] * pvec[i])
                    return 0

                lax.fori_loop(0, CH // 16, mul_body, 0)
                pltpu.sync_copy(fr, out_sp.at[dr], add=True)
                return 0

            lax.fori_loop(0, BLK, ch_a, 0)
            return 0

        lax.fori_loop(0, NBLK, blk_a, 0)
        plsc.subcore_barrier()

        pltpu.sync_copy(out_sp.at[pl.ds(rr, RPT)], outraw.at[c, pl.ds(rr, RPT)])
        for h in range(HEADS):
            pltpu.sync_copy(s_sp[h].at[pl.ds(rr, RPT)],
                            s_out.at[c, h, pl.ds(rr, RPT)])

    return k(src2d, dst2d, elT, erT, feat_p, z128, z1)


def _s1b(src2d, dst2d, elT, erT, sT):
    mesh = plsc.VectorSubcoreMesh(
        core_axis_name="c", subcore_axis_name="s", num_cores=NC, num_subcores=NS)

    @functools.partial(
        pl.kernel,
        mesh=mesh,
        out_type=jax.ShapeDtypeStruct((HEADS, E_PAD), jnp.float32),
        scratch_types=[
            pltpu.VMEM((BLK, CH), jnp.int32),
            pltpu.VMEM((BLK, CH), jnp.int32),
            pltpu.VMEM((HEADS * CH,), jnp.float32),
            pltpu.VMEM((HEADS * CH,), jnp.float32),
            pltpu.VMEM((HEADS * CH,), jnp.float32),
            pltpu.VMEM((HEADS * BLK * CH,), jnp.float32),
        ] + [pltpu.VMEM_SHARED((NPAD,), jnp.float32)] * 24
        + [pltpu.SemaphoreType.DMA],
    )
    def k(src_h, dst_h, elT_h, erT_h, sT_h, a_out,
          src_v, dst_v, el_b, er_b, s_b, a_st,
          e0, e1, e2, e3, e4, e5, e6, e7,
          r0_, r1, r2, r3, r4, r5, r6, r7,
          s0, s1, s2, s3, s4, s5, s6, s7,
          gsem):
        c = lax.axis_index("c")
        t = lax.axis_index("s")
        el_sp = [e0, e1, e2, e3, e4, e5, e6, e7]
        er_sp = [r0_, r1, r2, r3, r4, r5, r6, r7]
        s_sp = [s0, s1, s2, s3, s4, s5, s6, s7]

        rr = t * RPT
        for h in range(HEADS):
            pltpu.sync_copy(elT_h.at[h, pl.ds(rr, RPT)], el_sp[h].at[pl.ds(rr, RPT)])
            pltpu.sync_copy(erT_h.at[h, pl.ds(rr, RPT)], er_sp[h].at[pl.ds(rr, RPT)])
            pltpu.sync_copy(sT_h.at[h, pl.ds(rr, RPT)], s_sp[h].at[pl.ds(rr, RPT)])
        plsc.subcore_barrier()

        def blk_b(b, _):
            row0 = c * RPS + t * RPTE + b * BLK
            pltpu.sync_copy(src_h.at[pl.ds(row0, BLK)], src_v)
            pltpu.sync_copy(dst_h.at[pl.ds(row0, BLK)], dst_v)

            def ch_b(j, _2):
                sr = src_v.at[j]
                dr = dst_v.at[j]
                dl = []
                for h in range(HEADS):
                    dl.append(pltpu.async_copy(
                        el_sp[h].at[sr], el_b.at[pl.ds(h * CH, CH)], gsem))
                    dl.append(pltpu.async_copy(
                        er_sp[h].at[dr], er_b.at[pl.ds(h * CH, CH)], gsem))
                    dl.append(pltpu.async_copy(
                        s_sp[h].at[dr], s_b.at[pl.ds(h * CH, CH)], gsem))
                for d in dl:
                    d.wait()
                aoff = j * CH
                for h in range(HEADS):
                    for kk in range(CH // 16):
                        x = (el_b[pl.ds(h * CH + kk * 16, 16)]
                             + er_b[pl.ds(h * CH + kk * 16, 16)])
                        pv = jnp.exp(jnp.maximum(x, x * 0.2))
                        sv = s_b[pl.ds(h * CH + kk * 16, 16)]
                        a_st[pl.ds(h * BLK * CH + aoff + kk * 16, 16)] = (
                            pv / (sv + 1e-9))
                return 0

            lax.fori_loop(0, BLK, ch_b, 0)
            eb = (c * RPS + t * RPTE + b * BLK) * CH
            for h in range(HEADS):
                pltpu.sync_copy(a_st.at[pl.ds(h * BLK * CH, BLK * CH)],
                                a_out.at[h, pl.ds(eb, BLK * CH)])
            return 0

        lax.fori_loop(0, NBLK, blk_b, 0)

    return k(src2d, dst2d, elT, erT, sT)


def _t2_body(x_ref, s_ref, bh_ref, b_ref, w_ref, alr_ref, feat_ref, eler_ref):
    x = x_ref[0] + x_ref[1]
    s8 = s_ref[0] + s_ref[1]
    div = lax.dot_general(s8, bh_ref[...], (((0,), (0,)), ((), ())),
                          preferred_element_type=jnp.float32)
    h1 = x / (div + 1e-9) + b_ref[...]
    h1 = jnp.where(h1 > 0, h1, jnp.exp(jnp.minimum(h1, 0.0)) - 1.0)
    feat = jnp.dot(h1, w_ref[...], preferred_element_type=jnp.float32)
    feat_ref[...] = feat
    eler_ref[...] = jnp.dot(feat, alr_ref[...], preferred_element_type=jnp.float32)


def _t2(outraw, s_part, bhot, b1, W2, alr2):
    return pl.pallas_call(
        _t2_body,
        grid=(NPAD // 640,),
        in_specs=[
            pl.BlockSpec((NC, 640, F1), lambda i: (0, i, 0)),
            pl.BlockSpec((NC, HEADS, 640), lambda i: (0, 0, i)),
            pl.BlockSpec((HEADS, F1), lambda i: (0, 0)),
            pl.BlockSpec((1, F1), lambda i: (0, 0)),
            pl.BlockSpec((F1, OUT), lambda i: (0, 0)),
            pl.BlockSpec((OUT, 2), lambda i: (0, 0)),
        ],
        out_specs=[
            pl.BlockSpec((640, OUT), lambda i: (i, 0)),
            pl.BlockSpec((640, 2), lambda i: (i, 0)),
        ],
        out_shape=[
            jax.ShapeDtypeStruct((NPAD, OUT), jnp.float32),
            jax.ShapeDtypeStruct((NPAD, 2), jnp.float32),
        ],
    )(outraw, s_part, bhot, b1, W2, alr2)


def _s2(src2d, dst2d, el2, er2, feat2p, z128, z1):
    mesh = plsc.VectorSubcoreMesh(
        core_axis_name="c", subcore_axis_name="s", num_cores=NC, num_subcores=NS)

    @functools.partial(
        pl.kernel,
        mesh=mesh,
        out_type=[
            jax.ShapeDtypeStruct((NC, NPAD, F1), jnp.float32),
            jax.ShapeDtypeStruct((E_PAD,), jnp.float32),
        ],
        scratch_types=[
            pltpu.VMEM((BLK, CH), jnp.int32),
            pltpu.VMEM((BLK, CH), jnp.int32),
            pltpu.VMEM((CH,), jnp.float32),
            pltpu.VMEM((CH,), jnp.float32),
            pltpu.VMEM((CH,), jnp.float32),
            pltpu.VMEM((CH,), jnp.float32),
            pltpu.VMEM((CH, F1), jnp.float32),
            pltpu.VMEM((BLK * CH,), jnp.float32),
            pltpu.VMEM_SHARED((NPAD,), jnp.float32),
            pltpu.VMEM_SHARED((NPAD,), jnp.float32),
            pltpu.VMEM_SHARED((NPAD,), jnp.float32),
            pltpu.VMEM_SHARED((NPAD, F1), jnp.float32),
            pltpu.SemaphoreType.DMA,
            pltpu.SemaphoreType.DMA,
        ],
    )
    def k(src_h, dst_h, el_h, er_h, feat_h, z128_h, z1_h, out2raw, a_out,
          src_v, dst_v, el_b, er_b, s_b, p_b, fr2, a_st,
          el_sp, er_sp, s_sp, out_sp, gsem, fsem):
        c = lax.axis_index("c")
        t = lax.axis_index("s")

        rr = t * RPT
        pltpu.sync_copy(el_h.at[pl.ds(rr, RPT)], el_sp.at[pl.ds(rr, RPT)])
        pltpu.sync_copy(er_h.at[pl.ds(rr, RPT)], er_sp.at[pl.ds(rr, RPT)])
        pltpu.sync_copy(z1_h, s_sp.at[pl.ds(rr, RPT)])
        pltpu.sync_copy(z128_h, out_sp.at[pl.ds(rr, RPT)])
        plsc.subcore_barrier()

        def blk_a(b, _):
            row0 = t * (E_ROWS // NS) + b * BLK
            pltpu.sync_copy(src_h.at[pl.ds(row0, BLK)], src_v)
            pltpu.sync_copy(dst_h.at[pl.ds(row0, BLK)], dst_v)

            def ch_a(j, _2):
                sr = src_v.at[j]
                dr = dst_v.at[j]
                d1 = pltpu.async_copy(el_sp.at[sr], el_b, gsem)
                d2 = pltpu.async_copy(er_sp.at[dr], er_b, gsem)
                d1.wait()
                d2.wait()
                for kk in range(CH // 16):
                    x = el_b[pl.ds(kk * 16, 16)] + er_b[pl.ds(kk * 16, 16)]
                    p_b[pl.ds(kk * 16, 16)] = jnp.exp(jnp.maximum(x, x * 0.2))
                pltpu.sync_copy(p_b, s_sp.at[dr], add=True)
                return 0

            lax.fori_loop(0, BLK, ch_a, 0)
            return 0

        lax.fori_loop(0, E_ROWS // NS // BLK, blk_a, 0)
        plsc.subcore_barrier()

        def blk_b(b, _):
            row0 = c * RPS + t * RPTE + b * BLK
            pltpu.sync_copy(src_h.at[pl.ds(row0, BLK)], src_v)
            pltpu.sync_copy(dst_h.at[pl.ds(row0, BLK)], dst_v)

            def ch_b(j, _2):
                sr = src_v.at[j]
                dr = dst_v.at[j]
                d1 = pltpu.async_copy(el_sp.at[sr], el_b, gsem)
                d2 = pltpu.async_copy(er_sp.at[dr], er_b, gsem)
                d3 = pltpu.async_copy(s_sp.at[dr], s_b, gsem)
                df = pltpu.async_copy(feat_h.at[sr], fr2, fsem)
                d1.wait()
                d2.wait()
                d3.wait()
                aoff = j * CH
                for kk in range(CH // 16):
                    x = el_b[pl.ds(kk * 16, 16)] + er_b[pl.ds(kk * 16, 16)]
                    pv = jnp.exp(jnp.maximum(x, x * 0.2))
                    a_st[pl.ds(aoff + kk * 16, 16)] = (
                        pv / (s_b[pl.ds(kk * 16, 16)] + 1e-9))
                df.wait()

                def mul_body(g, _3):
                    gbase = g * 16
                    avec = a_st[pl.ds(aoff + gbase, 16)]
                    for i in range(16):
                        ei = gbase + i
                        fr2[ei, pl.ds(0, 16)] = fr2[ei, pl.ds(0, 16)] * avec[i]
                    return 0

                lax.fori_loop(0, CH // 16, mul_body, 0)
                pltpu.sync_copy(fr2, out_sp.at[dr], add=True)
                return 0

            lax.fori_loop(0, BLK, ch_b, 0)
            eb = (c * RPS + t * RPTE + b * BLK) * CH
            pltpu.sync_copy(a_st, a_out.at[pl.ds(eb, BLK * CH)])
            return 0

        lax.fori_loop(0, NBLK, blk_b, 0)
        plsc.subcore_barrier()

        pltpu.sync_copy(out_sp.at[pl.ds(rr, RPT)], out2raw.at[c, pl.ds(rr, RPT)])

    return k(src2d, dst2d, el2, er2, feat2p, z128, z1)


def _t3_body(x_ref, b_ref, o_ref):
    x = (x_ref[0] + x_ref[1])[:, :OUT] + b_ref[...]
    m = jnp.max(x, axis=1, keepdims=True)
    ex = jnp.exp(x - m)
    o_ref[...] = ex / jnp.sum(ex, axis=1, keepdims=True)


def _t3(out2raw, b2):
    return pl.pallas_call(
        _t3_body,
        grid=(N // ROWS,),
        in_specs=[
            pl.BlockSpec((NC, ROWS, F1), lambda i: (0, i, 0)),
            pl.BlockSpec((1, OUT), lambda i: (0, 0)),
        ],
        out_specs=pl.BlockSpec((ROWS, OUT), lambda i: (i, 0)),
        out_shape=jax.ShapeDtypeStruct((N, OUT), jnp.float32),
    )(out2raw, b2)


def _edge_phase(feat, el, er, src, dst):
    H = el.shape[1]
    D = feat.shape[1] // H
    e = jax.nn.leaky_relu(el[src] + er[dst], negative_slope=0.2)
    p = jnp.exp(e)
    s = jax.ops.segment_sum(p, dst, num_segments=N)
    a = p / (s[dst] + 1e-9)
    msg = feat[src].reshape(E, H, D) * a[:, :, None]
    out = jax.ops.segment_sum(msg.reshape(E, H * D), dst, num_segments=N)
    return a, out


def _block_diag_lr(al, ar):
    H, D = al.shape
    eye = jnp.eye(H, dtype=al.dtype)
    ml = (al[:, :, None] * eye[:, None, :]).reshape(H * D, H)
    mr = (ar[:, :, None] * eye[:, None, :]).reshape(H * D, H)
    return ml, mr


def kernel(h, edge_index, W1, al1, ar1, b1, W2, al2, ar2, b2):
    src = edge_index[0]
    dst = edge_index[1]

    ml1, mr1 = _block_diag_lr(al1, ar1)
    feat1, el1, er1 = _t1(h, W1, ml1, mr1)

    src_p = jnp.concatenate(
        [src, jnp.zeros((E_PAD - E,), jnp.int32)]).reshape(E_ROWS, CH)
    dst_p = jnp.concatenate(
        [dst, jnp.full((E_PAD - E,), N, jnp.int32)]).reshape(E_ROWS, CH)
    elT = jnp.pad(el1.T, ((0, 0), (0, NPAD - N)))
    erT = jnp.pad(er1.T, ((0, 0), (0, NPAD - N)))
    feat_p = jnp.pad(feat1, ((0, NPAD - N), (0, 0)))
    z128 = jnp.zeros((RPT, F1), jnp.float32)
    z1 = jnp.zeros((RPT,), jnp.float32)

    outraw, s_part = _s1a(src_p, dst_p, elT, erT, feat_p, z128, z1)
    sT = s_part[0] + s_part[1]
    a_raw = _s1b(src_p, dst_p, elT, erT, sT)
    a1 = a_raw[:, :E].T

    bhot = (jnp.eye(HEADS, dtype=jnp.float32)[:, :, None]
            * jnp.ones((1, 1, HID), jnp.float32)).reshape(HEADS, F1)
    alr2 = jnp.concatenate([al2.T, ar2.T], axis=1)
    feat2, eler2 = _t2(outraw, s_part, bhot, b1.reshape(1, -1), W2, alr2)

    feat2p = jnp.pad(feat2, ((0, 0), (0, F1 - OUT)))
    el2 = eler2[:, 0]
    er2 = eler2[:, 1]
    out2raw, a2_raw = _s2(src_p, dst_p, el2, er2, feat2p, z128, z1)
    a2 = a2_raw[:E].reshape(E, 1)

    out = _t3(out2raw, b2.reshape(1, -1))
    return out, [a1[:, :, None], a2[:, :, None]]


# speedup vs baseline: 30.6539x; 1.0148x over previous
import functools

import jax
import jax.numpy as jnp
from jax import lax
from jax.experimental import pallas as pl
from jax.experimental.pallas import tpu as pltpu
from jax.experimental.pallas import tpu_sc as plsc

N = 10000
E = 320000
IN_FEATS = 128
HID = 16
HEADS = 8
OUT = 16
F1 = HEADS * HID

ROWS = 1000

NC, NS = 2, 16
NPAD = 10240
RPT = NPAD // NS
CH = 128
BLK = 16
E_ROWS = 2560
E_PAD = E_ROWS * CH
RPS = E_ROWS // NC
RPTE = RPS // NS
NBLK = RPTE // BLK


def _t1_body(h_ref, w_ref, al_ref, ar_ref, feat_ref, el_ref, er_ref):
    feat = jnp.dot(h_ref[...], w_ref[...], preferred_element_type=jnp.float32)
    feat_ref[...] = feat
    el_ref[...] = jnp.dot(feat, al_ref[...], preferred_element_type=jnp.float32)
    er_ref[...] = jnp.dot(feat, ar_ref[...], preferred_element_type=jnp.float32)


def _t1(h, W1, al, ar):
    return pl.pallas_call(
        _t1_body,
        grid=(N // ROWS,),
        in_specs=[
            pl.BlockSpec((ROWS, IN_FEATS), lambda i: (i, 0)),
            pl.BlockSpec((IN_FEATS, F1), lambda i: (0, 0)),
            pl.BlockSpec((IN_FEATS, HEADS), lambda i: (0, 0)),
            pl.BlockSpec((IN_FEATS, HEADS), lambda i: (0, 0)),
        ],
        out_specs=[
            pl.BlockSpec((ROWS, F1), lambda i: (i, 0)),
            pl.BlockSpec((ROWS, HEADS), lambda i: (i, 0)),
            pl.BlockSpec((ROWS, HEADS), lambda i: (i, 0)),
        ],
        out_shape=[
            jax.ShapeDtypeStruct((N, F1), jnp.float32),
            jax.ShapeDtypeStruct((N, HEADS), jnp.float32),
            jax.ShapeDtypeStruct((N, HEADS), jnp.float32),
        ],
    )(h, W1, al, ar)


def _s1a(src2d, dst2d, elT, erT, feat_p, z128, z1):
    mesh = plsc.VectorSubcoreMesh(
        core_axis_name="c", subcore_axis_name="s", num_cores=NC, num_subcores=NS)

    @functools.partial(
        pl.kernel,
        mesh=mesh,
        out_type=[
            jax.ShapeDtypeStruct((NC, NPAD, F1), jnp.float32),
            jax.ShapeDtypeStruct((NC, HEADS, NPAD), jnp.float32),
        ],
        scratch_types=[
            pltpu.VMEM((BLK, CH), jnp.int32),
            pltpu.VMEM((BLK, CH), jnp.int32),
            pltpu.VMEM((HEADS * CH,), jnp.float32),
            pltpu.VMEM((HEADS * CH,), jnp.float32),
            pltpu.VMEM((HEADS * CH,), jnp.float32),
            pltpu.VMEM((CH, F1), jnp.float32),
        ] + [pltpu.VMEM_SHARED((NPAD,), jnp.float32)] * 24
        + [
            pltpu.VMEM_SHARED((NPAD, F1), jnp.float32),
            pltpu.SemaphoreType.DMA,
            pltpu.SemaphoreType.DMA,
            pltpu.SemaphoreType.DMA,
            pltpu.SemaphoreType.DMA,
        ],
    )
    def k(src_h, dst_h, elT_h, erT_h, feat_h, z128_h, z1_h, outraw, s_out,
          src_v, dst_v, el_b, er_b, p_b, fr,
          e0, e1, e2, e3, e4, e5, e6, e7,
          r0_, r1, r2, r3, r4, r5, r6, r7,
          s0, s1, s2, s3, s4, s5, s6, s7,
          out_sp, gsem, fsem, psem, osem):
        c = lax.axis_index("c")
        t = lax.axis_index("s")
        el_sp = [e0, e1, e2, e3, e4, e5, e6, e7]
        er_sp = [r0_, r1, r2, r3, r4, r5, r6, r7]
        s_sp = [s0, s1, s2, s3, s4, s5, s6, s7]

        rr = t * RPT
        for h in range(HEADS):
            pltpu.sync_copy(elT_h.at[h, pl.ds(rr, RPT)], el_sp[h].at[pl.ds(rr, RPT)])
            pltpu.sync_copy(erT_h.at[h, pl.ds(rr, RPT)], er_sp[h].at[pl.ds(rr, RPT)])
            pltpu.sync_copy(z1_h, s_sp[h].at[pl.ds(rr, RPT)])
        pltpu.sync_copy(z128_h, out_sp.at[pl.ds(rr, RPT)])
        plsc.subcore_barrier()

        def blk_a(b, _):
            row0 = c * RPS + t * RPTE + b * BLK
            pltpu.sync_copy(src_h.at[pl.ds(row0, BLK)], src_v)
            pltpu.sync_copy(dst_h.at[pl.ds(row0, BLK)], dst_v)

            def ch_a(j, _2):
                sr = src_v.at[j]
                dr = dst_v.at[j]
                dl = []
                for h in range(HEADS):
                    dl.append(pltpu.async_copy(
                        el_sp[h].at[sr], el_b.at[pl.ds(h * CH, CH)], gsem))
                    dl.append(pltpu.async_copy(
                        er_sp[h].at[dr], er_b.at[pl.ds(h * CH, CH)], gsem))
                df = pltpu.async_copy(feat_h.at[sr], fr, fsem)
                for d in dl:
                    d.wait()
                for h in range(HEADS):
                    for kk in range(CH // 16):
                        x = (el_b[pl.ds(h * CH + kk * 16, 16)]
                             + er_b[pl.ds(h * CH + kk * 16, 16)])
                        p_b[pl.ds(h * CH + kk * 16, 16)] = jnp.exp(
                            jnp.maximum(x, x * 0.2))
                pd = [pltpu.async_copy(p_b.at[pl.ds(h * CH, CH)],
                                       s_sp[h].at[dr], psem, add=True)
                      for h in range(HEADS)]
                df.wait()

                def mul_body(g, _3):
                    gbase = g * 16
                    for h in range(HEADS):
                        pvec = p_b[pl.ds(h * CH + gbase, 16)]
                        for i in range(16):
                            ei = gbase + i
                            fr[ei, pl.ds(h * 16, 16)] = (
                                fr[ei, pl.ds(h * 16, 16)] * pvec[i])
                    return 0

                lax.fori_loop(0, CH // 16, mul_body, 0)
                do = pltpu.async_copy(fr, out_sp.at[dr], osem, add=True)
                for d in pd:
                    d.wait()
                do.wait()
                return 0

            lax.fori_loop(0, BLK, ch_a, 0)
            return 0

        lax.fori_loop(0, NBLK, blk_a, 0)
        plsc.subcore_barrier()

        pltpu.sync_copy(out_sp.at[pl.ds(rr, RPT)], outraw.at[c, pl.ds(rr, RPT)])
        for h in range(HEADS):
            pltpu.sync_copy(s_sp[h].at[pl.ds(rr, RPT)],
                            s_out.at[c, h, pl.ds(rr, RPT)])

    return k(src2d, dst2d, elT, erT, feat_p, z128, z1)


def _s1b(src2d, dst2d, elT, erT, sT):
    mesh = plsc.VectorSubcoreMesh(
        core_axis_name="c", subcore_axis_name="s", num_cores=NC, num_subcores=NS)

    @functools.partial(
        pl.kernel,
        mesh=mesh,
        out_type=jax.ShapeDtypeStruct((HEADS, E_PAD), jnp.float32),
        scratch_types=[
            pltpu.VMEM((BLK, CH), jnp.int32),
            pltpu.VMEM((BLK, CH), jnp.int32),
            pltpu.VMEM((HEADS * CH,), jnp.float32),
            pltpu.VMEM((HEADS * CH,), jnp.float32),
            pltpu.VMEM((HEADS * CH,), jnp.float32),
            pltpu.VMEM((HEADS * BLK * CH,), jnp.float32),
        ] + [pltpu.VMEM_SHARED((NPAD,), jnp.float32)] * 24
        + [pltpu.SemaphoreType.DMA],
    )
    def k(src_h, dst_h, elT_h, erT_h, sT_h, a_out,
          src_v, dst_v, el_b, er_b, s_b, a_st,
          e0, e1, e2, e3, e4, e5, e6, e7,
          r0_, r1, r2, r3, r4, r5, r6, r7,
          s0, s1, s2, s3, s4, s5, s6, s7,
          gsem):
        c = lax.axis_index("c")
        t = lax.axis_index("s")
        el_sp = [e0, e1, e2, e3, e4, e5, e6, e7]
        er_sp = [r0_, r1, r2, r3, r4, r5, r6, r7]
        s_sp = [s0, s1, s2, s3, s4, s5, s6, s7]

        rr = t * RPT
        for h in range(HEADS):
            pltpu.sync_copy(elT_h.at[h, pl.ds(rr, RPT)], el_sp[h].at[pl.ds(rr, RPT)])
            pltpu.sync_copy(erT_h.at[h, pl.ds(rr, RPT)], er_sp[h].at[pl.ds(rr, RPT)])
            pltpu.sync_copy(sT_h.at[h, pl.ds(rr, RPT)], s_sp[h].at[pl.ds(rr, RPT)])
        plsc.subcore_barrier()

        def blk_b(b, _):
            row0 = c * RPS + t * RPTE + b * BLK
            pltpu.sync_copy(src_h.at[pl.ds(row0, BLK)], src_v)
            pltpu.sync_copy(dst_h.at[pl.ds(row0, BLK)], dst_v)

            def ch_b(j, _2):
                sr = src_v.at[j]
                dr = dst_v.at[j]
                dl = []
                for h in range(HEADS):
                    dl.append(pltpu.async_copy(
                        el_sp[h].at[sr], el_b.at[pl.ds(h * CH, CH)], gsem))
                    dl.append(pltpu.async_copy(
                        er_sp[h].at[dr], er_b.at[pl.ds(h * CH, CH)], gsem))
                    dl.append(pltpu.async_copy(
                        s_sp[h].at[dr], s_b.at[pl.ds(h * CH, CH)], gsem))
                for d in dl:
                    d.wait()
                aoff = j * CH
                for h in range(HEADS):
                    for kk in range(CH // 16):
                        x = (el_b[pl.ds(h * CH + kk * 16, 16)]
                             + er_b[pl.ds(h * CH + kk * 16, 16)])
                        pv = jnp.exp(jnp.maximum(x, x * 0.2))
                        sv = s_b[pl.ds(h * CH + kk * 16, 16)]
                        a_st[pl.ds(h * BLK * CH + aoff + kk * 16, 16)] = (
                            pv / (sv + 1e-9))
                return 0

            lax.fori_loop(0, BLK, ch_b, 0)
            eb = (c * RPS + t * RPTE + b * BLK) * CH
            for h in range(HEADS):
                pltpu.sync_copy(a_st.at[pl.ds(h * BLK * CH, BLK * CH)],
                                a_out.at[h, pl.ds(eb, BLK * CH)])
            return 0

        lax.fori_loop(0, NBLK, blk_b, 0)

    return k(src2d, dst2d, elT, erT, sT)


def _t2_body(x_ref, s_ref, bh_ref, b_ref, w_ref, alr_ref, feat_ref, eler_ref):
    x = x_ref[0] + x_ref[1]
    s8 = s_ref[0] + s_ref[1]
    div = lax.dot_general(s8, bh_ref[...], (((0,), (0,)), ((), ())),
                          preferred_element_type=jnp.float32)
    h1 = x / (div + 1e-9) + b_ref[...]
    h1 = jnp.where(h1 > 0, h1, jnp.exp(jnp.minimum(h1, 0.0)) - 1.0)
    feat = jnp.dot(h1, w_ref[...], preferred_element_type=jnp.float32)
    feat_ref[...] = feat
    eler_ref[...] = jnp.dot(feat, alr_ref[...], preferred_element_type=jnp.float32)


def _t2(outraw, s_part, bhot, b1, W2, alr2):
    return pl.pallas_call(
        _t2_body,
        grid=(NPAD // 640,),
        in_specs=[
            pl.BlockSpec((NC, 640, F1), lambda i: (0, i, 0)),
            pl.BlockSpec((NC, HEADS, 640), lambda i: (0, 0, i)),
            pl.BlockSpec((HEADS, F1), lambda i: (0, 0)),
            pl.BlockSpec((1, F1), lambda i: (0, 0)),
            pl.BlockSpec((F1, OUT), lambda i: (0, 0)),
            pl.BlockSpec((OUT, 2), lambda i: (0, 0)),
        ],
        out_specs=[
            pl.BlockSpec((640, OUT), lambda i: (i, 0)),
            pl.BlockSpec((640, 2), lambda i: (i, 0)),
        ],
        out_shape=[
            jax.ShapeDtypeStruct((NPAD, OUT), jnp.float32),
            jax.ShapeDtypeStruct((NPAD, 2), jnp.float32),
        ],
    )(outraw, s_part, bhot, b1, W2, alr2)


def _s2(src2d, dst2d, el2, er2, feat2p, z128, z1):
    mesh = plsc.VectorSubcoreMesh(
        core_axis_name="c", subcore_axis_name="s", num_cores=NC, num_subcores=NS)

    @functools.partial(
        pl.kernel,
        mesh=mesh,
        out_type=[
            jax.ShapeDtypeStruct((NC, NPAD, F1), jnp.float32),
            jax.ShapeDtypeStruct((E_PAD,), jnp.float32),
        ],
        scratch_types=[
            pltpu.VMEM((BLK, CH), jnp.int32),
            pltpu.VMEM((BLK, CH), jnp.int32),
            pltpu.VMEM((CH,), jnp.float32),
            pltpu.VMEM((CH,), jnp.float32),
            pltpu.VMEM((CH,), jnp.float32),
            pltpu.VMEM((CH,), jnp.float32),
            pltpu.VMEM((CH, F1), jnp.float32),
            pltpu.VMEM((BLK * CH,), jnp.float32),
            pltpu.VMEM_SHARED((NPAD,), jnp.float32),
            pltpu.VMEM_SHARED((NPAD,), jnp.float32),
            pltpu.VMEM_SHARED((NPAD,), jnp.float32),
            pltpu.VMEM_SHARED((NPAD, F1), jnp.float32),
            pltpu.SemaphoreType.DMA,
            pltpu.SemaphoreType.DMA,
        ],
    )
    def k(src_h, dst_h, el_h, er_h, feat_h, z128_h, z1_h, out2raw, a_out,
          src_v, dst_v, el_b, er_b, s_b, p_b, fr2, a_st,
          el_sp, er_sp, s_sp, out_sp, gsem, fsem):
        c = lax.axis_index("c")
        t = lax.axis_index("s")

        rr = t * RPT
        pltpu.sync_copy(el_h.at[pl.ds(rr, RPT)], el_sp.at[pl.ds(rr, RPT)])
        pltpu.sync_copy(er_h.at[pl.ds(rr, RPT)], er_sp.at[pl.ds(rr, RPT)])
        pltpu.sync_copy(z1_h, s_sp.at[pl.ds(rr, RPT)])
        pltpu.sync_copy(z128_h, out_sp.at[pl.ds(rr, RPT)])
        plsc.subcore_barrier()

        def blk_a(b, _):
            row0 = t * (E_ROWS // NS) + b * BLK
            pltpu.sync_copy(src_h.at[pl.ds(row0, BLK)], src_v)
            pltpu.sync_copy(dst_h.at[pl.ds(row0, BLK)], dst_v)

            def ch_a(j, _2):
                sr = src_v.at[j]
                dr = dst_v.at[j]
                d1 = pltpu.async_copy(el_sp.at[sr], el_b, gsem)
                d2 = pltpu.async_copy(er_sp.at[dr], er_b, gsem)
                d1.wait()
                d2.wait()
                for kk in range(CH // 16):
                    x = el_b[pl.ds(kk * 16, 16)] + er_b[pl.ds(kk * 16, 16)]
                    p_b[pl.ds(kk * 16, 16)] = jnp.exp(jnp.maximum(x, x * 0.2))
                pltpu.sync_copy(p_b, s_sp.at[dr], add=True)
                return 0

            lax.fori_loop(0, BLK, ch_a, 0)
            return 0

        lax.fori_loop(0, E_ROWS // NS // BLK, blk_a, 0)
        plsc.subcore_barrier()

        def blk_b(b, _):
            row0 = c * RPS + t * RPTE + b * BLK
            pltpu.sync_copy(src_h.at[pl.ds(row0, BLK)], src_v)
            pltpu.sync_copy(dst_h.at[pl.ds(row0, BLK)], dst_v)

            def ch_b(j, _2):
                sr = src_v.at[j]
                dr = dst_v.at[j]
                d1 = pltpu.async_copy(el_sp.at[sr], el_b, gsem)
                d2 = pltpu.async_copy(er_sp.at[dr], er_b, gsem)
                d3 = pltpu.async_copy(s_sp.at[dr], s_b, gsem)
                df = pltpu.async_copy(feat_h.at[sr], fr2, fsem)
                d1.wait()
                d2.wait()
                d3.wait()
                aoff = j * CH
                for kk in range(CH // 16):
                    x = el_b[pl.ds(kk * 16, 16)] + er_b[pl.ds(kk * 16, 16)]
                    pv = jnp.exp(jnp.maximum(x, x * 0.2))
                    a_st[pl.ds(aoff + kk * 16, 16)] = (
                        pv / (s_b[pl.ds(kk * 16, 16)] + 1e-9))
                df.wait()

                def mul_body(g, _3):
                    gbase = g * 16
                    avec = a_st[pl.ds(aoff + gbase, 16)]
                    for i in range(16):
                        ei = gbase + i
                        fr2[ei, pl.ds(0, 16)] = fr2[ei, pl.ds(0, 16)] * avec[i]
                    return 0

                lax.fori_loop(0, CH // 16, mul_body, 0)
                pltpu.sync_copy(fr2, out_sp.at[dr], add=True)
                return 0

            lax.fori_loop(0, BLK, ch_b, 0)
            eb = (c * RPS + t * RPTE + b * BLK) * CH
            pltpu.sync_copy(a_st, a_out.at[pl.ds(eb, BLK * CH)])
            return 0

        lax.fori_loop(0, NBLK, blk_b, 0)
        plsc.subcore_barrier()

        pltpu.sync_copy(out_sp.at[pl.ds(rr, RPT)], out2raw.at[c, pl.ds(rr, RPT)])

    return k(src2d, dst2d, el2, er2, feat2p, z128, z1)


def _t3_body(x_ref, b_ref, o_ref):
    x = (x_ref[0] + x_ref[1])[:, :OUT] + b_ref[...]
    m = jnp.max(x, axis=1, keepdims=True)
    ex = jnp.exp(x - m)
    o_ref[...] = ex / jnp.sum(ex, axis=1, keepdims=True)


def _t3(out2raw, b2):
    return pl.pallas_call(
        _t3_body,
        grid=(N // ROWS,),
        in_specs=[
            pl.BlockSpec((NC, ROWS, F1), lambda i: (0, i, 0)),
            pl.BlockSpec((1, OUT), lambda i: (0, 0)),
        ],
        out_specs=pl.BlockSpec((ROWS, OUT), lambda i: (i, 0)),
        out_shape=jax.ShapeDtypeStruct((N, OUT), jnp.float32),
    )(out2raw, b2)


def _edge_phase(feat, el, er, src, dst):
    H = el.shape[1]
    D = feat.shape[1] // H
    e = jax.nn.leaky_relu(el[src] + er[dst], negative_slope=0.2)
    p = jnp.exp(e)
    s = jax.ops.segment_sum(p, dst, num_segments=N)
    a = p / (s[dst] + 1e-9)
    msg = feat[src].reshape(E, H, D) * a[:, :, None]
    out = jax.ops.segment_sum(msg.reshape(E, H * D), dst, num_segments=N)
    return a, out


def _block_diag_lr(al, ar):
    H, D = al.shape
    eye = jnp.eye(H, dtype=al.dtype)
    ml = (al[:, :, None] * eye[:, None, :]).reshape(H * D, H)
    mr = (ar[:, :, None] * eye[:, None, :]).reshape(H * D, H)
    return ml, mr


def kernel(h, edge_index, W1, al1, ar1, b1, W2, al2, ar2, b2):
    src = edge_index[0]
    dst = edge_index[1]

    ml1, mr1 = _block_diag_lr(al1, ar1)
    feat1, el1, er1 = _t1(h, W1, ml1, mr1)

    src_p = jnp.concatenate(
        [src, jnp.zeros((E_PAD - E,), jnp.int32)]).reshape(E_ROWS, CH)
    dst_p = jnp.concatenate(
        [dst, jnp.full((E_PAD - E,), N, jnp.int32)]).reshape(E_ROWS, CH)
    elT = jnp.pad(el1.T, ((0, 0), (0, NPAD - N)))
    erT = jnp.pad(er1.T, ((0, 0), (0, NPAD - N)))
    feat_p = jnp.pad(feat1, ((0, NPAD - N), (0, 0)))
    z128 = jnp.zeros((RPT, F1), jnp.float32)
    z1 = jnp.zeros((RPT,), jnp.float32)

    outraw, s_part = _s1a(src_p, dst_p, elT, erT, feat_p, z128, z1)
    sT = s_part[0] + s_part[1]
    a_raw = _s1b(src_p, dst_p, elT, erT, sT)
    a1 = a_raw[:, :E].T

    bhot = (jnp.eye(HEADS, dtype=jnp.float32)[:, :, None]
            * jnp.ones((1, 1, HID), jnp.float32)).reshape(HEADS, F1)
    alr2 = jnp.concatenate([al2.T, ar2.T], axis=1)
    feat2, eler2 = _t2(outraw, s_part, bhot, b1.reshape(1, -1), W2, alr2)

    feat2p = jnp.pad(feat2, ((0, 0), (0, F1 - OUT)))
    el2 = eler2[:, 0]
    er2 = eler2[:, 1]
    out2raw, a2_raw = _s2(src_p, dst_p, el2, er2, feat2p, z128, z1)
    a2 = a2_raw[:E].reshape(E, 1)

    out = _t3(out2raw, b2.reshape(1, -1))
    return out, [a1[:, :, None], a2[:, :, None]]
